# baseline (speedup 1.0000x reference)
import functools
import jax
import jax.numpy as jnp
from jax import lax
from jax.experimental import pallas as pl
from jax.experimental.pallas import tpu as pltpu
from jax.experimental.pallas import tpu_sc as plsc

_FILL = 1e-15
_N = 32
_NC = 2
_NS = 16
_NW = _NC * _NS
_TILE = 512


def _top_kernel(x_ref, o_ref):
    x = x_ref[...]
    R, L = x.shape
    G = L // _N
    TOP = _N * _N // L

    lane = jax.lax.broadcasted_iota(jnp.int32, (R, L), 1)
    sub = lane & (_N - 1)

    v = x
    for s in (16, 8, 4, 2, 1):
        v = jnp.maximum(v, jnp.roll(v, -s, axis=1))
    for s in (1, 2, 4, 8, 16):
        v = jnp.where((sub & s) != 0, jnp.roll(v, s, axis=1), v)

    mi = jnp.where(x == v, sub, _N)
    for s in (16, 8, 4, 2, 1):
        mi = jnp.minimum(mi, jnp.roll(mi, -s, axis=1))
    for s in (1, 2, 4, 8, 16):
        mi = jnp.where((sub & s) != 0, jnp.roll(mi, s, axis=1), mi)

    oh = (mi == sub).astype(jnp.int32)
    red = jnp.max(oh, axis=0, keepdims=True)
    red = jnp.maximum(red, jnp.roll(red, 64, axis=1))
    red = jnp.maximum(red, jnp.roll(red, 32, axis=1))

    qi = jax.lax.broadcasted_iota(jnp.int32, (TOP, L), 0)
    ci = jax.lax.broadcasted_iota(jnp.int32, (TOP, L), 1)
    a = jnp.where(ci // G == qi, jnp.broadcast_to(red.astype(jnp.float32), (TOP, L)), 0.0)
    ri = jax.lax.broadcasted_iota(jnp.int32, (L, L), 0)
    li = jax.lax.broadcasted_iota(jnp.int32, (L, L), 1)
    b = jnp.where(ri % G == li // _N, 1.0, 0.0).astype(jnp.float32)
    keep = (
        jax.lax.dot_general(a, b, (((1,), (0,)), ((), ())),
                            preferred_element_type=jnp.float32)
        > 0.5
    )

    o_ref[...] = jnp.where(keep, x[:TOP, :], jnp.full((TOP, L), _FILL, jnp.float32))


def _dus_kernel(t_ref, f_ref, o_ref):
    o_ref[...] = t_ref[...]


def _make_fill(total_rows):
    rows_per_w = total_rows // _NW
    n_copies = rows_per_w // _TILE
    mesh = plsc.VectorSubcoreMesh(core_axis_name="c", subcore_axis_name="s")

    @functools.partial(
        pl.kernel,
        mesh=mesh,
        out_type=jax.ShapeDtypeStruct((total_rows, 128), jnp.float32),
        scratch_types=[
            pltpu.VMEM((_TILE, 128), jnp.float32),
            pltpu.SemaphoreType.DMA,
        ],
    )
    def fill_k(tile_hbm, out_hbm, tile_v, sem):
        wid = lax.axis_index("s") * _NC + lax.axis_index("c")
        base = wid * rows_per_w
        pltpu.sync_copy(tile_hbm, tile_v)
        handles = []
        for i in range(n_copies):
            handles.append(
                pltpu.async_copy(tile_v, out_hbm.at[pl.ds(base + i * _TILE, _TILE)], sem)
            )
        for h in handles:
            h.wait()

    return fill_k


@jax.jit
def kernel(X):
    B, M, N = X.shape
    R = M * N // 128
    TOP = N * N // 128
    total = B * R
    Xv = X.reshape(total, 128)

    tops = pl.pallas_call(
        _top_kernel,
        grid=(B,),
        in_specs=[pl.BlockSpec((R, 128), lambda i: (i, 0))],
        out_specs=pl.BlockSpec((TOP, 128), lambda i: (i, 0)),
        out_shape=jax.ShapeDtypeStruct((B * TOP, 128), jnp.float32),
    )(Xv)

    tile = jnp.full((_TILE, 128), _FILL, jnp.float32)
    filled = _make_fill(total)(tile)

    nb = R // TOP
    out = pl.pallas_call(
        _dus_kernel,
        grid=(B,),
        in_specs=[
            pl.BlockSpec((TOP, 128), lambda i: (i, 0)),
            pl.BlockSpec(memory_space=pl.ANY),
        ],
        out_specs=pl.BlockSpec((TOP, 128), lambda i: (i * nb, 0)),
        out_shape=jax.ShapeDtypeStruct((total, 128), jnp.float32),
        input_output_aliases={1: 0},
    )(tops, filled)
    return out.reshape(B, M, N)


# speedup vs baseline: 1.0245x; 1.0245x over previous
import functools
import jax
import jax.numpy as jnp
from jax import lax
from jax.experimental import pallas as pl
from jax.experimental.pallas import tpu as pltpu
from jax.experimental.pallas import tpu_sc as plsc

_FILL = 1e-15
_N = 32
_NC = 2
_NS = 16
_NW = _NC * _NS


def _top_kernel(x_ref, o_ref):
    x = x_ref[...]
    R, L = x.shape
    G = L // _N
    TOP = _N * _N // L

    lane = jax.lax.broadcasted_iota(jnp.int32, (R, L), 1)
    sub = lane & (_N - 1)

    v = x
    for s in (16, 8, 4, 2, 1):
        v = jnp.maximum(v, jnp.roll(v, -s, axis=1))
    for s in (1, 2, 4, 8, 16):
        v = jnp.where((sub & s) != 0, jnp.roll(v, s, axis=1), v)

    mi = jnp.where(x == v, sub, _N)
    for s in (16, 8, 4, 2, 1):
        mi = jnp.minimum(mi, jnp.roll(mi, -s, axis=1))
    for s in (1, 2, 4, 8, 16):
        mi = jnp.where((sub & s) != 0, jnp.roll(mi, s, axis=1), mi)

    oh = (mi == sub).astype(jnp.int32)
    red = jnp.max(oh, axis=0, keepdims=True)
    red = jnp.maximum(red, jnp.roll(red, 64, axis=1))
    red = jnp.maximum(red, jnp.roll(red, 32, axis=1))

    qi = jax.lax.broadcasted_iota(jnp.int32, (TOP, L), 0)
    ci = jax.lax.broadcasted_iota(jnp.int32, (TOP, L), 1)
    a = jnp.where(ci // G == qi, jnp.broadcast_to(red.astype(jnp.float32), (TOP, L)), 0.0)
    ri = jax.lax.broadcasted_iota(jnp.int32, (L, L), 0)
    li = jax.lax.broadcasted_iota(jnp.int32, (L, L), 1)
    b = jnp.where(ri % G == li // _N, 1.0, 0.0).astype(jnp.float32)
    keep = (
        jax.lax.dot_general(a, b, (((1,), (0,)), ((), ())),
                            preferred_element_type=jnp.float32)
        > 0.5
    )

    o_ref[...] = jnp.where(keep, x[:TOP, :], jnp.full((TOP, L), _FILL, jnp.float32))


def _make_assemble(total_rows, batch_rows, top_rows):
    nbatch = total_rows // batch_rows
    b_per_w = nbatch // _NW
    fill_rows = batch_rows - top_rows
    n_fill = 5
    tile = fill_rows // n_fill
    assert tile % 8 == 0 and tile * n_fill == fill_rows
    mesh = plsc.VectorSubcoreMesh(core_axis_name="c", subcore_axis_name="s")

    @functools.partial(
        pl.kernel,
        mesh=mesh,
        out_type=jax.ShapeDtypeStruct((total_rows, 128), jnp.float32),
        scratch_types=[
            pltpu.VMEM((tile, 128), jnp.float32),
            pltpu.VMEM((top_rows, 128), jnp.float32),
            pltpu.SemaphoreType.DMA,
        ],
    )
    def assemble_k(tile_hbm, tops_hbm, out_hbm, tile_v, top_v, sem):
        wid = lax.axis_index("s") * _NC + lax.axis_index("c")
        pltpu.sync_copy(tile_hbm, tile_v)
        handles = []
        for j in range(b_per_w):
            b = wid * b_per_w + j
            base = b * batch_rows
            pltpu.sync_copy(tops_hbm.at[pl.ds(b * top_rows, top_rows)], top_v)
            pltpu.sync_copy(top_v, out_hbm.at[pl.ds(base, top_rows)])
            for i in range(n_fill):
                handles.append(
                    pltpu.async_copy(
                        tile_v,
                        out_hbm.at[pl.ds(base + top_rows + i * tile, tile)],
                        sem,
                    )
                )
        for h in handles:
            h.wait()

    return assemble_k


@jax.jit
def kernel(X):
    B, M, N = X.shape
    R = M * N // 128
    TOP = N * N // 128
    total = B * R
    Xv = X.reshape(total, 128)

    tops = pl.pallas_call(
        _top_kernel,
        grid=(B,),
        in_specs=[pl.BlockSpec((R, 128), lambda i: (i, 0))],
        out_specs=pl.BlockSpec((TOP, 128), lambda i: (i, 0)),
        out_shape=jax.ShapeDtypeStruct((B * TOP, 128), jnp.float32),
    )(Xv)

    fill_tile = jnp.full(((R - TOP) // 5, 128), _FILL, jnp.float32)
    out = _make_assemble(total, R, TOP)(fill_tile, tops)
    return out.reshape(B, M, N)


# speedup vs baseline: 1.2710x; 1.2406x over previous
import jax
import jax.numpy as jnp
from jax.experimental import pallas as pl
from jax.experimental.pallas import tpu as pltpu

_FILL = 1e-15


def _compute(x):
    M, N = x.shape
    iota = jax.lax.broadcasted_iota(jnp.int32, (M, N), 1)
    rmax = jnp.max(x, axis=1, keepdims=True)
    ismax = x == rmax
    idx = jnp.min(jnp.where(ismax, iota, N), axis=1, keepdims=True)
    onehot = (iota == idx).astype(jnp.float32)
    cnt = jax.lax.dot_general(
        onehot,
        jnp.ones((M, 1), jnp.float32),
        (((0,), (0,)), ((), ())),
        preferred_element_type=jnp.float32,
    )
    keep = cnt > 0.5
    top = jnp.where(keep, x[:N, :], jnp.full((N, N), _FILL, jnp.float32))
    return top


def _mk_kernel(B, M, N):
    def body(x_hbm, o_hbm, xv0, xv1, ov0, ov1, si0, si1, so0, so1):
        i = pl.program_id(0)
        xv = [xv0, xv1]
        ov = [ov0, ov1]
        si = [si0, si1]
        so = [so0, so1]

        for p in range(2):
            @pl.when((i < B) & (i % 2 == p))
            def _():
                pltpu.make_async_copy(x_hbm.at[i], xv[p], si[p]).start()

            @pl.when((i >= 1) & (i <= B) & ((i - 1) % 2 == p))
            def _():
                j = i - 1
                pltpu.make_async_copy(x_hbm.at[j], xv[p], si[p]).wait()

                @pl.when(j >= 2)
                def _():
                    pltpu.make_async_copy(ov[p], o_hbm.at[j - 2], so[p]).wait()

                x = xv[p][...]
                ov[p][...] = jnp.full((M, N), _FILL, jnp.float32)
                ov[p][:N, :] = _compute(x)
                pltpu.make_async_copy(ov[p], o_hbm.at[j], so[p]).start()

            @pl.when(i == B + 1)
            def _():
                pltpu.make_async_copy(ov[p], o_hbm.at[B - 2 + p], so[p]).wait()

    return body


@jax.jit
def kernel(X):
    B, M, N = X.shape
    return pl.pallas_call(
        _mk_kernel(B, M, N),
        grid=(B + 2,),
        in_specs=[pl.BlockSpec(memory_space=pl.ANY)],
        out_specs=pl.BlockSpec(memory_space=pl.ANY),
        out_shape=jax.ShapeDtypeStruct((B, M, N), jnp.float32),
        scratch_shapes=[
            pltpu.VMEM((M, N), jnp.float32),
            pltpu.VMEM((M, N), jnp.float32),
            pltpu.VMEM((M, N), jnp.float32),
            pltpu.VMEM((M, N), jnp.float32),
            pltpu.SemaphoreType.DMA,
            pltpu.SemaphoreType.DMA,
            pltpu.SemaphoreType.DMA,
            pltpu.SemaphoreType.DMA,
        ],
    )(X)


# speedup vs baseline: 1.5144x; 1.1915x over previous
import jax
import jax.numpy as jnp
from jax.experimental import pallas as pl

_FILL = 1e-15


def _max_layer_kernel(x_ref, o_ref):
    x = x_ref[0]
    M, N = x.shape
    iota = jax.lax.broadcasted_iota(jnp.int32, (M, N), 1)
    rmax = jnp.max(x, axis=1, keepdims=True)
    ismax = x == rmax
    idx = jnp.min(jnp.where(ismax, iota, N), axis=1, keepdims=True)
    onehot = (iota == idx).astype(jnp.float32)
    cnt = jax.lax.dot_general(
        onehot,
        jnp.ones((M, 1), jnp.float32),
        (((0,), (0,)), ((), ())),
        preferred_element_type=jnp.float32,
    )
    keep = cnt > 0.5
    o_ref[0] = jnp.full((M, N), _FILL, dtype=jnp.float32)
    o_ref[0, :N, :] = jnp.where(keep, x[:N, :], jnp.full((N, N), _FILL, jnp.float32))


@jax.jit
def kernel(X):
    B, M, N = X.shape
    return pl.pallas_call(
        _max_layer_kernel,
        grid=(B,),
        in_specs=[pl.BlockSpec((1, M, N), lambda i: (i, 0, 0))],
        out_specs=pl.BlockSpec((1, M, N), lambda i: (i, 0, 0)),
        out_shape=jax.ShapeDtypeStruct((B, M, N), jnp.float32),
    )(X)


# speedup vs baseline: 1.7673x; 1.1670x over previous
import jax
import jax.numpy as jnp
from jax.experimental import pallas as pl

_FILL = 1e-15


def _max_layer_kernel(x_ref, o_ref):
    x = x_ref[0]
    M, N = x.shape
    iota = jax.lax.broadcasted_iota(jnp.int32, (M, N), 1)
    idx = jnp.argmax(x, axis=1, keepdims=True).astype(jnp.int32)
    onehot = (iota == idx).astype(jnp.float32)
    cnt = jax.lax.dot_general(
        onehot,
        jnp.ones((M, 1), jnp.float32),
        (((0,), (0,)), ((), ())),
        preferred_element_type=jnp.float32,
    )
    keep = cnt > 0.5
    o_ref[0] = jnp.full((M, N), _FILL, dtype=jnp.float32)
    o_ref[0, :N, :] = jnp.where(keep, x[:N, :], jnp.full((N, N), _FILL, jnp.float32))


@jax.jit
def kernel(X):
    B, M, N = X.shape
    return pl.pallas_call(
        _max_layer_kernel,
        grid=(B,),
        in_specs=[pl.BlockSpec((1, M, N), lambda i: (i, 0, 0))],
        out_specs=pl.BlockSpec((1, M, N), lambda i: (i, 0, 0)),
        out_shape=jax.ShapeDtypeStruct((B, M, N), jnp.float32),
    )(X)
